# baseline (speedup 1.0000x reference)
import jax
import jax.numpy as jnp
from jax import lax
from jax.experimental import pallas as pl
from jax.experimental.pallas import tpu as pltpu
from jax.experimental.pallas import tpu_sc as plsc

EMB_ROWS = 1000000
EMB_DIM = 32
BATCH = 16384
NUM_CORES = 2
NUM_SUBCORES = 16
LANES = 16
NUM_WORKERS = NUM_CORES * NUM_SUBCORES
ROWS_PER_WORKER = BATCH // NUM_WORKERS
CHUNK = 128
NCHUNKS = ROWS_PER_WORKER // CHUNK
GROUPS = CHUNK // LANES


def _mf_body(p1_hbm, p2_hbm, table_hbm, out_hbm,
             idx1_v, idx2_v, rows1_v, rows2_v, out_v,
             sem1a, sem1b, sem2a, sem2b):
    wid = lax.axis_index("s") * NUM_CORES + lax.axis_index("c")
    base = wid * ROWS_PER_WORKER

    pltpu.sync_copy(p1_hbm.at[pl.ds(base, ROWS_PER_WORKER)], idx1_v)
    pltpu.sync_copy(p2_hbm.at[pl.ds(base, ROWS_PER_WORKER)], idx2_v)

    sems1 = (sem1a, sem1b)
    sems2 = (sem2a, sem2b)

    def start_chunk(c, buf):
        def issue(g, carry):
            iv1 = idx1_v[pl.ds(c * CHUNK + g * LANES, LANES)]
            iv2 = idx2_v[pl.ds(c * CHUNK + g * LANES, LANES)]
            for r in range(LANES):
                slot = g * LANES + r
                pltpu.async_copy(table_hbm.at[pl.ds(iv1[r], 1)],
                                 rows1_v.at[buf, pl.ds(slot, 1)], sems1[buf])
                pltpu.async_copy(table_hbm.at[pl.ds(iv2[r], 1)],
                                 rows2_v.at[buf, pl.ds(slot, 1)], sems2[buf])
            return carry
        lax.fori_loop(0, GROUPS, issue, 0)

    def wait_chunk(buf):
        def drain(r, carry):
            pltpu.make_async_copy(table_hbm.at[pl.ds(0, 1)],
                                  rows1_v.at[buf, pl.ds(0, 1)],
                                  sems1[buf]).wait()
            pltpu.make_async_copy(table_hbm.at[pl.ds(0, 1)],
                                  rows2_v.at[buf, pl.ds(0, 1)],
                                  sems2[buf]).wait()
            return carry
        lax.fori_loop(0, CHUNK, drain, 0)

    lane = lax.iota(jnp.int32, LANES)

    def shuffle(v, perm):
        return lax.gather(
            v, perm[:, None],
            lax.GatherDimensionNumbers(
                offset_dims=(), collapsed_slice_dims=(0,),
                start_index_map=(0,)),
            slice_sizes=(1,),
            mode=lax.GatherScatterMode.PROMISE_IN_BOUNDS)

    def combine(a, b, k):
        m = (lane & k) == 0
        sel_ab = jnp.where(m, a, b)
        sel_ba = jnp.where(m, b, a)
        return sel_ab + shuffle(sel_ba, lane ^ k)

    def compute_chunk(buf, out_base):
        r1 = rows1_v.at[buf]
        r2 = rows2_v.at[buf]

        def group(g, carry):
            row0 = g * LANES
            w = []
            for r in range(LANES):
                row = row0 + r
                a0 = r1[row, pl.ds(0, LANES)]
                a1 = r1[row, pl.ds(LANES, LANES)]
                b0 = r2[row, pl.ds(0, LANES)]
                b1 = r2[row, pl.ds(LANES, LANES)]
                w.append(a0 * b0 + a1 * b1)
            for k in (1, 2, 4, 8):
                w = [combine(w[2 * i], w[2 * i + 1], k)
                     for i in range(len(w) // 2)]
            acc = w[0]
            out_v[pl.ds(out_base + row0, LANES)] = 1.0 / (1.0 + jnp.exp(-acc))
            return carry

        lax.fori_loop(0, GROUPS, group, 0)

    start_chunk(0, 0)
    for c in range(NCHUNKS):
        buf = c % 2
        if c + 1 < NCHUNKS:
            start_chunk(c + 1, 1 - buf)
        wait_chunk(buf)
        compute_chunk(buf, c * CHUNK)

    pltpu.sync_copy(out_v, out_hbm.at[pl.ds(base, ROWS_PER_WORKER)])


def kernel(product1, product2, embedding_weight):
    mesh = plsc.VectorSubcoreMesh(core_axis_name="c", subcore_axis_name="s")
    run = pl.kernel(
        _mf_body,
        mesh=mesh,
        out_type=jax.ShapeDtypeStruct((BATCH,), jnp.float32),
        scratch_types=[
            pltpu.VMEM((ROWS_PER_WORKER,), jnp.int32),
            pltpu.VMEM((ROWS_PER_WORKER,), jnp.int32),
            pltpu.VMEM((2, CHUNK, EMB_DIM), jnp.float32),
            pltpu.VMEM((2, CHUNK, EMB_DIM), jnp.float32),
            pltpu.VMEM((ROWS_PER_WORKER,), jnp.float32),
            pltpu.SemaphoreType.DMA,
            pltpu.SemaphoreType.DMA,
            pltpu.SemaphoreType.DMA,
            pltpu.SemaphoreType.DMA,
        ],
    )
    return run(product1.astype(jnp.int32), product2.astype(jnp.int32),
               embedding_weight)
